# baseline (speedup 1.0000x reference)
import functools

import jax
import jax.numpy as jnp
from jax import lax
from jax.experimental import pallas as pl
from jax.experimental.pallas import tpu as pltpu
from jax.experimental.pallas import tpu_sc as plsc

DIM = 64
B = 16384

_NC = 2
_NS = 16
_NW = _NC * _NS
_B_PER_W = B // _NW
_CHUNK = 128
_NCHUNK = _B_PER_W // _CHUNK


@functools.lru_cache(maxsize=None)
def _build_gather():
    mesh = plsc.VectorSubcoreMesh(core_axis_name="c", subcore_axis_name="s")

    @functools.partial(
        pl.kernel,
        mesh=mesh,
        out_type=jax.ShapeDtypeStruct((B, DIM), jnp.float32),
        scratch_types=[
            pltpu.VMEM((_NCHUNK, _CHUNK), jnp.int32),
            pltpu.VMEM((_B_PER_W, DIM), jnp.float32),
            pltpu.SemaphoreType.DMA,
        ],
        compiler_params=pltpu.CompilerParams(use_tc_tiling_on_sc=False),
    )
    def gather(idx_hbm, table_hbm, out_hbm, idx_v, rows_v, sem):
        wid = lax.axis_index("s") * _NC + lax.axis_index("c")
        base = wid * _B_PER_W
        pltpu.sync_copy(idx_hbm.at[wid], idx_v)
        copies = []
        for j in range(_NCHUNK):
            copies.append(
                pltpu.async_copy(
                    table_hbm.at[idx_v.at[j]],
                    rows_v.at[pl.ds(j * _CHUNK, _CHUNK)],
                    sem,
                )
            )
        for c in copies:
            c.wait()
        pltpu.sync_copy(rows_v, out_hbm.at[pl.ds(base, _B_PER_W)])

    return gather


def kernel(head, entity, rel):
    idx = head.astype(jnp.int32).reshape(_NW, _NCHUNK, _CHUNK)
    return _build_gather()(idx, entity)


# speedup vs baseline: 6.6248x; 6.6248x over previous
import functools

import jax
import jax.numpy as jnp
from jax import lax
from jax.experimental import pallas as pl
from jax.experimental.pallas import tpu as pltpu
from jax.experimental.pallas import tpu_sc as plsc

VOCAB_E = 1000000
DIM = 64
B = 16384

_NC = 2
_NS = 16
_NW = _NC * _NS
_B_PER_W = B // _NW
_LANES = 16
_G = 8
_NGROUP = _B_PER_W // _LANES


@functools.lru_cache(maxsize=None)
def _build_gather():
    mesh = plsc.VectorSubcoreMesh(core_axis_name="c", subcore_axis_name="s")

    @functools.partial(
        pl.kernel,
        mesh=mesh,
        out_type=jax.ShapeDtypeStruct((8, 8, B), jnp.float32),
        scratch_types=[
            pltpu.VMEM((_B_PER_W,), jnp.int32),
            pltpu.VMEM((8, 8, _LANES * _G), jnp.float32),
            pltpu.VMEM((8, 8, _B_PER_W), jnp.float32),
            pltpu.SemaphoreType.DMA,
            pltpu.SemaphoreType.DMA,
        ],
        compiler_params=pltpu.CompilerParams(
            use_tc_tiling_on_sc=True, needs_layout_passes=False),
    )
    def gather(idx_hbm, t3_hbm, out_hbm, idx_v, win_v, buf_v, sem_i, sem):
        wid = lax.axis_index("s") * _NC + lax.axis_index("c")
        pltpu.async_copy(idx_hbm.at[wid], idx_v, sem_i).wait()
        lane_iota = lax.iota(jnp.int32, _LANES)

        def group(g, _):
            vec = idx_v[pl.ds(g * _LANES, _LANES)]
            a_vec = jnp.bitwise_and(vec, -_G)
            l_vec = jnp.bitwise_and(vec, _G - 1)
            for j in range(_LANES):
                a = pl.multiple_of(jnp.sum(jnp.where(lane_iota == j, a_vec, 0)), _G)
                pltpu.async_copy(
                    t3_hbm.at[:, :, pl.ds(a, _G)],
                    win_v.at[:, :, pl.ds(j * _G, _G)],
                    sem,
                )
            pltpu.make_async_copy(
                t3_hbm.at[:, :, pl.ds(0, _LANES * _G)],
                buf_v.at[:, :, pl.ds(0, _LANES * _G)],
                sem,
            ).wait()
            base = g * _LANES
            minor_idx = lane_iota * _G + l_vec
            for dblk in range(8):
                for sub in range(8):
                    sel = plsc.load_gather(
                        win_v,
                        [
                            jnp.full((_LANES,), dblk, jnp.int32),
                            jnp.full((_LANES,), sub, jnp.int32),
                            minor_idx,
                        ],
                    )
                    buf_v[dblk, sub, pl.ds(base, _LANES)] = sel
            return ()

        lax.fori_loop(0, _NGROUP, group, ())
        base = wid * _B_PER_W
        pltpu.async_copy(
            buf_v, out_hbm.at[:, :, pl.ds(base, _B_PER_W)], sem_i).wait()

    return gather


def kernel(head, entity, rel):
    idx = head.astype(jnp.int32).reshape(_NW, _B_PER_W)
    t3 = entity.T.reshape(8, 8, VOCAB_E)
    out3 = _build_gather()(idx, t3)
    return out3.reshape(DIM, B).T
